# baseline (speedup 1.0000x reference)
import jax
import jax.numpy as jnp
from jax import lax
from jax.experimental import pallas as pl
from jax.experimental.pallas import tpu as pltpu
from jax.experimental.pallas import tpu_sc as plsc

N = 50000
D = 128
H = 64
K = 5
T = 64
TC_COLS = T * K
R = 1000
NB = N // R

NW = 32
CHUNK = 1568
NGRP = CHUNK // 16

_NT = (((1,), (1,)), ((), ()))
_NN = (((1,), (0,)), ((), ()))


def _sc_hist_kernel(ids_hbm, out_hbm, chunk_v):
    wid = lax.axis_index("s") * 2 + lax.axis_index("c")
    base = jnp.minimum(wid * CHUNK, N - CHUNK)
    pltpu.sync_copy(ids_hbm.at[pl.ds(base, CHUNK)], chunk_v)
    acc_lo = jnp.zeros((16,), jnp.int32)
    acc_hi = jnp.zeros((16,), jnp.int32)
    one16 = jnp.ones((16,), jnp.int32)
    zero16 = jnp.zeros((16,), jnp.int32)
    for j in range(NGRP):
        v = chunk_v[pl.ds(16 * j, 16)]
        bit = lax.shift_left(one16, v & 31)
        acc_lo = acc_lo | jnp.where(v < 32, bit, zero16)
        acc_hi = acc_hi | jnp.where(v >= 32, bit, zero16)
    chunk_v[pl.ds(0, 16)] = acc_lo
    chunk_v[pl.ds(16, 16)] = acc_hi
    zero16 = jnp.zeros((16,), jnp.int32)
    for k in range(2, 8):
        chunk_v[pl.ds(16 * k, 16)] = zero16
    pltpu.sync_copy(chunk_v.at[pl.ds(0, 128)], out_hbm.at[wid])


def _sc_hist(ids):
    run = pl.kernel(
        _sc_hist_kernel,
        mesh=plsc.VectorSubcoreMesh(core_axis_name="c", subcore_axis_name="s"),
        out_type=jax.ShapeDtypeStruct((NW, 128), jnp.int32),
        scratch_types=[
            pltpu.VMEM((CHUNK,), jnp.int32),
        ],
    )
    return run(ids)


def _main_kernel(x_ref, ids_ref, tbl_ref, w1_ref, b1_ref, w2_ref, b2_ref,
                 s_ref, cid_ref, m_ref, rank_ref):
    @pl.when(pl.program_id(0) == 0)
    def _build_lut():
        masks = tbl_ref[...]
        acc = masks[0:1, :]
        for w in range(1, NW):
            acc = acc | masks[w:w + 1, :]
        lo = acc[:, 0:16]
        hi = acc[:, 16:32]
        for half in (8, 4, 2, 1):
            lo = lo[:, 0:half] | lo[:, half:2 * half]
            hi = hi[:, 0:half] | hi[:, half:2 * half]
        lane = lax.broadcasted_iota(jnp.int32, (1, T), 1)
        word = jnp.where(lane < 32, lo, hi)
        pres_row = ((lax.shift_right_logical(word, lane & 31) & 1)
                    ).astype(jnp.float32)
        ri = lax.broadcasted_iota(jnp.int32, (T, T), 0)
        ci = lax.broadcasted_iota(jnp.int32, (T, T), 1)
        diag = jnp.where(ri == ci, pres_row, 0.0)
        ones_col = jnp.zeros((T, 1), jnp.float32) + 1.0
        pres_col = lax.dot_general(diag, ones_col, _NN,
                                   preferred_element_type=jnp.float32)
        ltri = (ci < ri).astype(jnp.float32)
        rank = lax.dot_general(ltri, pres_col, _NN,
                               preferred_element_type=jnp.float32)
        rank_ref[...] = rank
        ranki = rank.astype(jnp.int32)
        colb = lax.broadcasted_iota(jnp.int32, (T, TC_COLS), 1) // K
        m_ref[...] = (colb == ranki).astype(jnp.float32)

    x = x_ref[...]
    h = lax.dot_general(x, w1_ref[...], _NT,
                        preferred_element_type=jnp.float32)
    h = jnp.maximum(h + b1_ref[...], 0.0)
    logits = lax.dot_general(h, w2_ref[...], _NT,
                             preferred_element_type=jnp.float32)
    logits = logits + b2_ref[...]
    mx = jnp.max(logits, axis=1, keepdims=True)
    e = jnp.exp(logits - mx)
    sl = e / jnp.sum(e, axis=1, keepdims=True)

    ids_row = ids_ref[...].reshape(1, R)
    ids = jnp.transpose(ids_row, (1, 0))
    onehot = (ids == lax.broadcasted_iota(jnp.int32, (1, T), 1)
              ).astype(jnp.float32)
    row_mask = lax.dot_general(onehot, m_ref[...], _NN,
                               preferred_element_type=jnp.float32)

    pj = lax.broadcasted_iota(jnp.int32, (K, TC_COLS), 0)
    pc = lax.broadcasted_iota(jnp.int32, (K, TC_COLS), 1)
    p = (pc % K == pj).astype(jnp.float32)
    tiled = lax.dot_general(sl, p, _NN,
                            preferred_element_type=jnp.float32)
    s_ref[...] = row_mask * tiled

    ranks = lax.dot_general(onehot, rank_ref[...], _NN,
                            preferred_element_type=jnp.float32)
    lane = lax.broadcasted_iota(jnp.int32, (1, K), 1).astype(jnp.float32)
    cand = jnp.where(sl == jnp.max(sl, axis=1, keepdims=True), lane,
                     jnp.float32(K))
    am = jnp.min(cand, axis=1, keepdims=True)
    cid_f = ranks * K + am
    cid_row = jnp.transpose(cid_f, (1, 0)).astype(jnp.int32)
    cid_ref[...] = cid_row.reshape(1, 1, R)


def kernel(x, transformer_ids, W1, b1, W2, b2):
    ids = transformer_ids.astype(jnp.int32)
    ids3 = ids.reshape(NB, 1, R)
    b1r = b1.reshape(1, H)
    b2r = b2.reshape(1, K)

    tbl = _sc_hist(ids)

    s, cid = pl.pallas_call(
        _main_kernel,
        grid=(NB,),
        in_specs=[
            pl.BlockSpec((R, D), lambda i: (i, 0)),
            pl.BlockSpec((1, 1, R), lambda i: (i, 0, 0)),
            pl.BlockSpec((NW, 128), lambda i: (0, 0)),
            pl.BlockSpec((H, D), lambda i: (0, 0)),
            pl.BlockSpec((1, H), lambda i: (0, 0)),
            pl.BlockSpec((K, H), lambda i: (0, 0)),
            pl.BlockSpec((1, K), lambda i: (0, 0)),
        ],
        out_specs=[
            pl.BlockSpec((R, TC_COLS), lambda i: (i, 0)),
            pl.BlockSpec((1, 1, R), lambda i: (i, 0, 0)),
        ],
        out_shape=[
            jax.ShapeDtypeStruct((N, TC_COLS), jnp.float32),
            jax.ShapeDtypeStruct((NB, 1, R), jnp.int32),
        ],
        scratch_shapes=[
            pltpu.VMEM((T, TC_COLS), jnp.float32),
            pltpu.VMEM((T, 1), jnp.float32),
        ],
    )(x, ids3, tbl, W1, b1r, W2, b2r)

    return (s, cid.reshape(N))


# speedup vs baseline: 1.1166x; 1.1166x over previous
import jax
import jax.numpy as jnp
from jax import lax
from jax.experimental import pallas as pl
from jax.experimental.pallas import tpu as pltpu
from jax.experimental.pallas import tpu_sc as plsc

N = 50000
D = 128
H = 64
K = 5
T = 64
TC_COLS = T * K
R = 2000
NB = N // R

NW = 32
CHUNK = 1568
NGRP = CHUNK // 16

_NT = (((1,), (1,)), ((), ()))
_NN = (((1,), (0,)), ((), ()))


def _sc_hist_kernel(ids_hbm, out_hbm, chunk_v):
    wid = lax.axis_index("s") * 2 + lax.axis_index("c")
    base = jnp.minimum(wid * CHUNK, N - CHUNK)
    pltpu.sync_copy(ids_hbm.at[pl.ds(base, CHUNK)], chunk_v)
    acc_lo = jnp.zeros((16,), jnp.int32)
    acc_hi = jnp.zeros((16,), jnp.int32)
    one16 = jnp.ones((16,), jnp.int32)
    zero16 = jnp.zeros((16,), jnp.int32)
    for j in range(NGRP):
        v = chunk_v[pl.ds(16 * j, 16)]
        bit = lax.shift_left(one16, v & 31)
        acc_lo = acc_lo | jnp.where(v < 32, bit, zero16)
        acc_hi = acc_hi | jnp.where(v >= 32, bit, zero16)
    chunk_v[pl.ds(0, 16)] = acc_lo
    chunk_v[pl.ds(16, 16)] = acc_hi
    zero16 = jnp.zeros((16,), jnp.int32)
    for k in range(2, 8):
        chunk_v[pl.ds(16 * k, 16)] = zero16
    pltpu.sync_copy(chunk_v.at[pl.ds(0, 128)], out_hbm.at[wid])


def _sc_hist(ids):
    run = pl.kernel(
        _sc_hist_kernel,
        mesh=plsc.VectorSubcoreMesh(core_axis_name="c", subcore_axis_name="s"),
        out_type=jax.ShapeDtypeStruct((NW, 128), jnp.int32),
        scratch_types=[
            pltpu.VMEM((CHUNK,), jnp.int32),
        ],
    )
    return run(ids)


def _main_kernel(x_ref, ids_ref, tbl_ref, w1_ref, b1_ref, w2_ref, b2_ref,
                 s_ref, cid_ref, m_ref, rank_ref):
    @pl.when(pl.program_id(0) == 0)
    def _build_lut():
        masks = tbl_ref[...]
        acc = masks[0:1, :]
        for w in range(1, NW):
            acc = acc | masks[w:w + 1, :]
        lo = acc[:, 0:16]
        hi = acc[:, 16:32]
        for half in (8, 4, 2, 1):
            lo = lo[:, 0:half] | lo[:, half:2 * half]
            hi = hi[:, 0:half] | hi[:, half:2 * half]
        lane = lax.broadcasted_iota(jnp.int32, (1, T), 1)
        word = jnp.where(lane < 32, lo, hi)
        pres_row = ((lax.shift_right_logical(word, lane & 31) & 1)
                    ).astype(jnp.float32)
        ri = lax.broadcasted_iota(jnp.int32, (T, T), 0)
        ci = lax.broadcasted_iota(jnp.int32, (T, T), 1)
        diag = jnp.where(ri == ci, pres_row, 0.0)
        ones_col = jnp.zeros((T, 1), jnp.float32) + 1.0
        pres_col = lax.dot_general(diag, ones_col, _NN,
                                   preferred_element_type=jnp.float32)
        ltri = (ci < ri).astype(jnp.float32)
        rank = lax.dot_general(ltri, pres_col, _NN,
                               preferred_element_type=jnp.float32)
        rank_ref[...] = rank
        ranki = rank.astype(jnp.int32)
        colb = lax.broadcasted_iota(jnp.int32, (T, TC_COLS), 1) // K
        m_ref[...] = (colb == ranki).astype(jnp.float32)

    x = x_ref[...]
    h = lax.dot_general(x, w1_ref[...], _NT,
                        preferred_element_type=jnp.float32)
    h = jnp.maximum(h + b1_ref[...], 0.0)
    logits = lax.dot_general(h, w2_ref[...], _NT,
                             preferred_element_type=jnp.float32)
    logits = logits + b2_ref[...]
    mx = jnp.max(logits, axis=1, keepdims=True)
    e = jnp.exp(logits - mx)
    sl = e / jnp.sum(e, axis=1, keepdims=True)

    ids_row = ids_ref[...].reshape(1, R)
    ids = jnp.transpose(ids_row, (1, 0))
    onehot = (ids == lax.broadcasted_iota(jnp.int32, (1, T), 1)
              ).astype(jnp.float32)
    row_mask = lax.dot_general(onehot, m_ref[...], _NN,
                               preferred_element_type=jnp.float32)

    pj = lax.broadcasted_iota(jnp.int32, (K, TC_COLS), 0)
    pc = lax.broadcasted_iota(jnp.int32, (K, TC_COLS), 1)
    p = (pc % K == pj).astype(jnp.float32)
    tiled = lax.dot_general(sl, p, _NN,
                            preferred_element_type=jnp.float32)
    s_ref[...] = row_mask * tiled

    ranks = lax.dot_general(onehot, rank_ref[...], _NN,
                            preferred_element_type=jnp.float32)
    lane = lax.broadcasted_iota(jnp.int32, (1, K), 1).astype(jnp.float32)
    cand = jnp.where(sl == jnp.max(sl, axis=1, keepdims=True), lane,
                     jnp.float32(K))
    am = jnp.min(cand, axis=1, keepdims=True)
    cid_f = ranks * K + am
    cid_row = jnp.transpose(cid_f, (1, 0)).astype(jnp.int32)
    cid_ref[...] = cid_row.reshape(1, 1, R)


def kernel(x, transformer_ids, W1, b1, W2, b2):
    ids = transformer_ids.astype(jnp.int32)
    ids3 = ids.reshape(NB, 1, R)
    b1r = b1.reshape(1, H)
    b2r = b2.reshape(1, K)

    tbl = _sc_hist(ids)

    s, cid = pl.pallas_call(
        _main_kernel,
        grid=(NB,),
        in_specs=[
            pl.BlockSpec((R, D), lambda i: (i, 0)),
            pl.BlockSpec((1, 1, R), lambda i: (i, 0, 0)),
            pl.BlockSpec((NW, 128), lambda i: (0, 0)),
            pl.BlockSpec((H, D), lambda i: (0, 0)),
            pl.BlockSpec((1, H), lambda i: (0, 0)),
            pl.BlockSpec((K, H), lambda i: (0, 0)),
            pl.BlockSpec((1, K), lambda i: (0, 0)),
        ],
        out_specs=[
            pl.BlockSpec((R, TC_COLS), lambda i: (i, 0)),
            pl.BlockSpec((1, 1, R), lambda i: (i, 0, 0)),
        ],
        out_shape=[
            jax.ShapeDtypeStruct((N, TC_COLS), jnp.float32),
            jax.ShapeDtypeStruct((NB, 1, R), jnp.int32),
        ],
        scratch_shapes=[
            pltpu.VMEM((T, TC_COLS), jnp.float32),
            pltpu.VMEM((T, 1), jnp.float32),
        ],
    )(x, ids3, tbl, W1, b1r, W2, b2r)

    return (s, cid.reshape(N))
